# Initial kernel scaffold; baseline (speedup 1.0000x reference)
import jax
import jax.numpy as jnp
from jax.experimental import pallas as pl


def kernel(x, edge_index, W1, b1, W2, b2, W3, b3):
    raise NotImplementedError("write your pallas kernel here")


# speedup vs baseline: 18.0087x; 18.0087x over previous
import functools

import jax
import jax.numpy as jnp
from jax import lax
from jax.experimental import pallas as pl
from jax.experimental.pallas import tpu as pltpu
from jax.experimental.pallas import tpu_sc as plsc

N = 10000
E = 640000
NC = 2
NS = 16
NW = NC * NS
BLK = 128
N_PAD = 10112
STRIPE = N_PAD // NS

NBLK16 = 158
E_PAD16 = NW * NBLK16 * BLK
NBLK64 = 313
E_PAD64 = NS * NBLK64 * BLK

R = 2000

_SC_PARAMS = pltpu.CompilerParams(use_tc_tiling_on_sc=False)


def _make_agg16():
  mesh = plsc.VectorSubcoreMesh(core_axis_name="c", subcore_axis_name="s")
  out_type = jax.ShapeDtypeStruct((NC, N_PAD, 16), jnp.float32)
  scratch = [
      pltpu.VMEM((NBLK16, BLK), jnp.int32),
      pltpu.VMEM((NBLK16, BLK), jnp.int32),
      pltpu.VMEM((BLK, 16), jnp.float32),
      pltpu.VMEM_SHARED((N_PAD, 16), jnp.float32),
      pltpu.SemaphoreType.DMA,
  ]

  @functools.partial(pl.kernel, out_type=out_type, mesh=mesh,
                     scratch_types=scratch, compiler_params=_SC_PARAMS)
  def agg(tbl_hbm, src_hbm, dst_hbm, zeros_hbm, out_hbm,
          src_v, dst_v, rows_v, acc, sem):
    cid = lax.axis_index("c")
    sid = lax.axis_index("s")
    wid = sid * NC + cid
    pltpu.sync_copy(zeros_hbm, acc.at[pl.ds(sid * STRIPE, STRIPE)])
    pltpu.sync_copy(src_hbm.at[wid], src_v)
    pltpu.sync_copy(dst_hbm.at[wid], dst_v)
    plsc.subcore_barrier()

    def body(j, carry):
      pltpu.async_copy(tbl_hbm.at[src_v.at[j]], rows_v, sem).wait()
      pltpu.sync_copy(rows_v, acc.at[dst_v.at[j]], add=True)
      return carry

    lax.fori_loop(0, NBLK16, body, 0)
    plsc.subcore_barrier()
    pltpu.sync_copy(acc.at[pl.ds(sid * STRIPE, STRIPE)],
                    out_hbm.at[cid, pl.ds(sid * STRIPE, STRIPE)])

  return agg


def _make_agg64():
  mesh = plsc.VectorSubcoreMesh(core_axis_name="c", subcore_axis_name="s")
  out_type = jax.ShapeDtypeStruct((NC, N_PAD, 64), jnp.float32)
  scratch = [
      pltpu.VMEM((NBLK64, BLK), jnp.int32),
      pltpu.VMEM((NBLK64, BLK), jnp.int32),
      pltpu.VMEM((BLK, 64), jnp.float32),
      pltpu.VMEM_SHARED((N_PAD, 64), jnp.float32),
      pltpu.SemaphoreType.DMA,
  ]

  @functools.partial(pl.kernel, out_type=out_type, mesh=mesh,
                     scratch_types=scratch, compiler_params=_SC_PARAMS)
  def agg(tbl_hbm, src_hbm, dst_hbm, zeros_hbm, out_hbm,
          src_v, dst_v, rows_v, acc, sem):
    cid = lax.axis_index("c")
    sid = lax.axis_index("s")
    pltpu.sync_copy(zeros_hbm, acc.at[pl.ds(sid * STRIPE, STRIPE)])
    pltpu.sync_copy(src_hbm.at[cid, sid], src_v)
    pltpu.sync_copy(dst_hbm.at[sid], dst_v)
    plsc.subcore_barrier()

    def body(j, carry):
      pltpu.async_copy(tbl_hbm.at[src_v.at[j]], rows_v, sem).wait()
      pltpu.sync_copy(rows_v, acc.at[dst_v.at[j]], add=True)
      return carry

    lax.fori_loop(0, NBLK64, body, 0)
    plsc.subcore_barrier()
    pltpu.sync_copy(acc.at[pl.ds(sid * STRIPE, STRIPE)],
                    out_hbm.at[cid, pl.ds(sid * STRIPE, STRIPE)])

  return agg


_agg16 = _make_agg16()
_agg64 = _make_agg64()


def _row_spec(w):
  return pl.BlockSpec((R, w), lambda i: (i, 0))


def _full_spec(shape):
  return pl.BlockSpec(shape, lambda i: tuple(0 for _ in shape))


def _tc_first(p0, p1, x, w1):
  k_in = x.shape[1]

  def body(p0_ref, p1_ref, x_ref, w_ref, dinv_ref, t_ref):
    deg = 1.0 + p0_ref[...] + p1_ref[...]
    dinv = lax.rsqrt(deg)
    dinv_ref[...] = dinv
    y = dinv * jnp.dot(x_ref[...], w_ref[...],
                       preferred_element_type=jnp.float32,
                       precision=lax.Precision.HIGHEST)
    t_ref[0] = y[:, :64]
    t_ref[1] = y[:, 64:]

  return pl.pallas_call(
      body,
      grid=(N // R,),
      in_specs=[_row_spec(1), _row_spec(1), _row_spec(k_in),
                _full_spec((k_in, 128))],
      out_specs=[_row_spec(1), pl.BlockSpec((2, R, 64), lambda i: (0, i, 0))],
      out_shape=[jax.ShapeDtypeStruct((N, 1), jnp.float32),
                 jax.ShapeDtypeStruct((2, N, 64), jnp.float32)],
  )(p0, p1, x, w1)


def _tc_mid(t, q0, q1, dinv, b, w_next, split_out):
  w_out = w_next.shape[1]

  def body(t_ref, q0_ref, q1_ref, dinv_ref, b_ref, w_ref, o_ref):
    dinv = dinv_ref[...]
    s = jnp.concatenate(
        [t_ref[0] + q0_ref[...], t_ref[1] + q1_ref[...]], axis=1)
    h = jnp.maximum(dinv * s + b_ref[...], 0.0)
    o = dinv * jnp.dot(h, w_ref[...],
                       preferred_element_type=jnp.float32,
                       precision=lax.Precision.HIGHEST)
    if split_out:
      o_ref[0] = o[:, :64]
      o_ref[1] = o[:, 64:]
    else:
      o_ref[...] = o

  if split_out:
    out_spec = pl.BlockSpec((2, R, 64), lambda i: (0, i, 0))
    out_shape = jax.ShapeDtypeStruct((2, N, 64), jnp.float32)
  else:
    out_spec = _row_spec(w_out)
    out_shape = jax.ShapeDtypeStruct((N, w_out), jnp.float32)

  return pl.pallas_call(
      body,
      grid=(N // R,),
      in_specs=[pl.BlockSpec((2, R, 64), lambda i: (0, i, 0)),
                _row_spec(64), _row_spec(64), _row_spec(1),
                _full_spec((1, 128)), _full_spec((128, w_out))],
      out_specs=out_spec,
      out_shape=out_shape,
  )(t, q0, q1, dinv, b, w_next)


def _tc_last(y, r0, r1, dinv, b):
  def body(y_ref, r0_ref, r1_ref, dinv_ref, b_ref, o_ref):
    o_ref[...] = (dinv_ref[...] * (y_ref[...] + r0_ref[...] + r1_ref[...])
                  + b_ref[...])

  return pl.pallas_call(
      body,
      grid=(N // R,),
      in_specs=[_row_spec(16), _row_spec(16), _row_spec(16), _row_spec(1),
                _full_spec((1, 16))],
      out_specs=_row_spec(16),
      out_shape=jax.ShapeDtypeStruct((N, 16), jnp.float32),
  )(y, r0, r1, dinv, b)


def kernel(x, edge_index, W1, b1, W2, b2, W3, b3):
  src = edge_index[0]
  dst = edge_index[1]

  p16 = E_PAD16 - E
  a16 = jnp.arange(p16, dtype=jnp.int32)
  src16 = jnp.concatenate([src, (a16 * 37) % N]).reshape(NW, NBLK16, BLK)
  dst16 = jnp.concatenate([dst, N + (a16 % (N_PAD - N))]).reshape(
      NW, NBLK16, BLK)

  p64 = E_PAD64 - E
  a64 = jnp.arange(p64, dtype=jnp.int32)
  srcf = jnp.concatenate([src, (a64 * 37) % N])
  dstf = jnp.concatenate([dst, N + (a64 % (N_PAD - N))])
  src64 = jnp.stack([srcf, srcf + N]).reshape(NC, NS, NBLK64, BLK)
  dst64 = dstf.reshape(NS, NBLK64, BLK)

  zeros16 = jnp.zeros((STRIPE, 16), jnp.float32)
  zeros64 = jnp.zeros((STRIPE, 64), jnp.float32)
  ones16 = jnp.ones((N, 16), jnp.float32)

  dp = _agg16(ones16, src16, dst16, zeros16)
  dinv, t1 = _tc_first(dp[0, :N, 0:1], dp[1, :N, 0:1], x, W1)

  p = _agg64(t1.reshape(NC * N, 64), src64, dst64, zeros64)
  t2 = _tc_mid(t1, p[0, :N], p[1, :N], dinv, b1.reshape(1, 128), W2, True)

  p = _agg64(t2.reshape(NC * N, 64), src64, dst64, zeros64)
  w3p = jnp.pad(W3, ((0, 0), (0, 16 - W3.shape[1])))
  y3 = _tc_mid(t2, p[0, :N], p[1, :N], dinv, b2.reshape(1, 128), w3p, False)

  r = _agg16(y3, src16, dst16, zeros16)
  b3p = jnp.pad(b3, (0, 16 - b3.shape[0])).reshape(1, 16)
  out16 = _tc_last(y3, r[0, :N], r[1, :N], dinv, b3p)
  return out16[:, :2]


# speedup vs baseline: 24.2681x; 1.3476x over previous
import functools

import jax
import jax.numpy as jnp
from jax import lax
from jax.experimental import pallas as pl
from jax.experimental.pallas import tpu as pltpu
from jax.experimental.pallas import tpu_sc as plsc

N = 10000
E = 640000
NC = 2
NS = 16
NW = NC * NS
BLK = 128
N_PAD = 10112
STRIPE = N_PAD // NS

NBLK16 = 158
E_PAD16 = NW * NBLK16 * BLK
WBLK = 158
NWIN = 2
NBLK64 = NWIN * WBLK
E_PAD64 = NS * NBLK64 * BLK

R = 2000

_SC_PARAMS = pltpu.CompilerParams(use_tc_tiling_on_sc=False)


def _agg_loop(nblk, tbl_hbm, src_v, dst_v, buf_a, buf_b, acc, sem_a, sem_b):
  pltpu.async_copy(tbl_hbm.at[src_v.at[0]], buf_a, sem_a)
  half = nblk // 2

  def body(j2, carry):
    j = 2 * j2
    pltpu.make_async_copy(tbl_hbm.at[src_v.at[j]], buf_a, sem_a).wait()
    pltpu.async_copy(tbl_hbm.at[src_v.at[j + 1]], buf_b, sem_b)
    pltpu.sync_copy(buf_a, acc.at[dst_v.at[j]], add=True)
    pltpu.make_async_copy(tbl_hbm.at[src_v.at[j + 1]], buf_b, sem_b).wait()

    @pl.when(j2 + 1 < half)
    def _():
      pltpu.async_copy(tbl_hbm.at[src_v.at[j + 2]], buf_a, sem_a)

    pltpu.sync_copy(buf_b, acc.at[dst_v.at[j + 1]], add=True)
    return carry

  lax.fori_loop(0, half, body, 0)


def _make_agg16():
  mesh = plsc.VectorSubcoreMesh(core_axis_name="c", subcore_axis_name="s")
  out_type = jax.ShapeDtypeStruct((NC, N_PAD, 16), jnp.float32)
  scratch = [
      pltpu.VMEM((NBLK16, BLK), jnp.int32),
      pltpu.VMEM((NBLK16, BLK), jnp.int32),
      pltpu.VMEM((BLK, 16), jnp.float32),
      pltpu.VMEM((BLK, 16), jnp.float32),
      pltpu.VMEM_SHARED((N_PAD, 16), jnp.float32),
      pltpu.SemaphoreType.DMA,
      pltpu.SemaphoreType.DMA,
  ]

  @functools.partial(pl.kernel, out_type=out_type, mesh=mesh,
                     scratch_types=scratch, compiler_params=_SC_PARAMS)
  def agg(tbl_hbm, src_hbm, dst_hbm, zeros_hbm, out_hbm,
          src_v, dst_v, buf_a, buf_b, acc, sem_a, sem_b):
    cid = lax.axis_index("c")
    sid = lax.axis_index("s")
    wid = sid * NC + cid
    pltpu.sync_copy(zeros_hbm, acc.at[pl.ds(sid * STRIPE, STRIPE)])
    pltpu.sync_copy(src_hbm.at[wid], src_v)
    pltpu.sync_copy(dst_hbm.at[wid], dst_v)
    plsc.subcore_barrier()
    _agg_loop(NBLK16, tbl_hbm, src_v, dst_v, buf_a, buf_b, acc,
              sem_a, sem_b)
    plsc.subcore_barrier()
    pltpu.sync_copy(acc.at[pl.ds(sid * STRIPE, STRIPE)],
                    out_hbm.at[cid, pl.ds(sid * STRIPE, STRIPE)])

  return agg


def _make_agg64():
  mesh = plsc.VectorSubcoreMesh(core_axis_name="c", subcore_axis_name="s")
  out_type = jax.ShapeDtypeStruct((NC, N_PAD, 64), jnp.float32)
  scratch = [
      pltpu.VMEM((WBLK, BLK), jnp.int32),
      pltpu.VMEM((WBLK, BLK), jnp.int32),
      pltpu.VMEM((BLK, 64), jnp.float32),
      pltpu.VMEM((BLK, 64), jnp.float32),
      pltpu.VMEM_SHARED((N_PAD, 64), jnp.float32),
      pltpu.SemaphoreType.DMA,
      pltpu.SemaphoreType.DMA,
  ]

  @functools.partial(pl.kernel, out_type=out_type, mesh=mesh,
                     scratch_types=scratch, compiler_params=_SC_PARAMS)
  def agg(tbl_hbm, src_hbm, dst_hbm, zeros_hbm, out_hbm,
          src_v, dst_v, buf_a, buf_b, acc, sem_a, sem_b):
    cid = lax.axis_index("c")
    sid = lax.axis_index("s")
    pltpu.sync_copy(zeros_hbm, acc.at[pl.ds(sid * STRIPE, STRIPE)])
    plsc.subcore_barrier()

    def win(w, carry):
      pltpu.sync_copy(src_hbm.at[cid, sid, pl.ds(w * WBLK, WBLK)], src_v)
      pltpu.sync_copy(dst_hbm.at[sid, pl.ds(w * WBLK, WBLK)], dst_v)
      _agg_loop(WBLK, tbl_hbm, src_v, dst_v, buf_a, buf_b, acc,
                sem_a, sem_b)
      return carry

    lax.fori_loop(0, NWIN, win, 0)
    plsc.subcore_barrier()
    pltpu.sync_copy(acc.at[pl.ds(sid * STRIPE, STRIPE)],
                    out_hbm.at[cid, pl.ds(sid * STRIPE, STRIPE)])

  return agg


def _make_deg16():
  mesh = plsc.VectorSubcoreMesh(core_axis_name="c", subcore_axis_name="s")
  out_type = jax.ShapeDtypeStruct((NC, N_PAD, 16), jnp.float32)
  scratch = [
      pltpu.VMEM((NBLK16, BLK), jnp.int32),
      pltpu.VMEM((BLK, 16), jnp.float32),
      pltpu.VMEM_SHARED((N_PAD, 16), jnp.float32),
  ]

  @functools.partial(pl.kernel, out_type=out_type, mesh=mesh,
                     scratch_types=scratch, compiler_params=_SC_PARAMS)
  def deg(ones_hbm, dst_hbm, zeros_hbm, out_hbm, dst_v, ones_v, acc):
    cid = lax.axis_index("c")
    sid = lax.axis_index("s")
    wid = sid * NC + cid
    pltpu.sync_copy(zeros_hbm, acc.at[pl.ds(sid * STRIPE, STRIPE)])
    pltpu.sync_copy(dst_hbm.at[wid], dst_v)
    pltpu.sync_copy(ones_hbm, ones_v)
    plsc.subcore_barrier()

    def body(j, carry):
      pltpu.sync_copy(ones_v, acc.at[dst_v.at[j]], add=True)
      return carry

    lax.fori_loop(0, NBLK16, body, 0)
    plsc.subcore_barrier()
    pltpu.sync_copy(acc.at[pl.ds(sid * STRIPE, STRIPE)],
                    out_hbm.at[cid, pl.ds(sid * STRIPE, STRIPE)])

  return deg


_agg16 = _make_agg16()
_agg64 = _make_agg64()
_deg16 = _make_deg16()


def _row_spec(w):
  return pl.BlockSpec((R, w), lambda i: (i, 0))


def _full_spec(shape):
  return pl.BlockSpec(shape, lambda i: tuple(0 for _ in shape))


def _tc_first(p0, p1, x, w1):
  k_in = x.shape[1]

  def body(p0_ref, p1_ref, x_ref, w_ref, dinv_ref, t_ref):
    deg = 1.0 + p0_ref[...] + p1_ref[...]
    dinv = lax.rsqrt(deg)
    dinv_ref[...] = dinv
    y = dinv * jnp.dot(x_ref[...], w_ref[...],
                       preferred_element_type=jnp.float32,
                       precision=lax.Precision.HIGHEST)
    t_ref[0] = y[:, :64]
    t_ref[1] = y[:, 64:]

  return pl.pallas_call(
      body,
      grid=(N // R,),
      in_specs=[_row_spec(1), _row_spec(1), _row_spec(k_in),
                _full_spec((k_in, 128))],
      out_specs=[_row_spec(1), pl.BlockSpec((2, R, 64), lambda i: (0, i, 0))],
      out_shape=[jax.ShapeDtypeStruct((N, 1), jnp.float32),
                 jax.ShapeDtypeStruct((2, N, 64), jnp.float32)],
  )(p0, p1, x, w1)


def _tc_mid(t, q0, q1, dinv, b, w_next, split_out):
  w_out = w_next.shape[1]

  def body(t_ref, q0_ref, q1_ref, dinv_ref, b_ref, w_ref, o_ref):
    dinv = dinv_ref[...]
    s = jnp.concatenate(
        [t_ref[0] + q0_ref[...], t_ref[1] + q1_ref[...]], axis=1)
    h = jnp.maximum(dinv * s + b_ref[...], 0.0)
    o = dinv * jnp.dot(h, w_ref[...],
                       preferred_element_type=jnp.float32,
                       precision=lax.Precision.HIGHEST)
    if split_out:
      o_ref[0] = o[:, :64]
      o_ref[1] = o[:, 64:]
    else:
      o_ref[...] = o

  if split_out:
    out_spec = pl.BlockSpec((2, R, 64), lambda i: (0, i, 0))
    out_shape = jax.ShapeDtypeStruct((2, N, 64), jnp.float32)
  else:
    out_spec = _row_spec(w_out)
    out_shape = jax.ShapeDtypeStruct((N, w_out), jnp.float32)

  return pl.pallas_call(
      body,
      grid=(N // R,),
      in_specs=[pl.BlockSpec((2, R, 64), lambda i: (0, i, 0)),
                _row_spec(64), _row_spec(64), _row_spec(1),
                _full_spec((1, 128)), _full_spec((128, w_out))],
      out_specs=out_spec,
      out_shape=out_shape,
  )(t, q0, q1, dinv, b, w_next)


def _tc_last(y, r0, r1, dinv, b):
  def body(y_ref, r0_ref, r1_ref, dinv_ref, b_ref, o_ref):
    o_ref[...] = (dinv_ref[...] * (y_ref[...] + r0_ref[...] + r1_ref[...])
                  + b_ref[...])

  return pl.pallas_call(
      body,
      grid=(N // R,),
      in_specs=[_row_spec(16), _row_spec(16), _row_spec(16), _row_spec(1),
                _full_spec((1, 16))],
      out_specs=_row_spec(16),
      out_shape=jax.ShapeDtypeStruct((N, 16), jnp.float32),
  )(y, r0, r1, dinv, b)


def kernel(x, edge_index, W1, b1, W2, b2, W3, b3):
  src = edge_index[0]
  dst = edge_index[1]

  p16 = E_PAD16 - E
  a16 = jnp.arange(p16, dtype=jnp.int32)
  src16 = jnp.concatenate([src, (a16 * 37) % N]).reshape(NW, NBLK16, BLK)
  dst16 = jnp.concatenate([dst, N + (a16 % (N_PAD - N))]).reshape(
      NW, NBLK16, BLK)

  p64 = E_PAD64 - E
  a64 = jnp.arange(p64, dtype=jnp.int32)
  srcf = jnp.concatenate([src, (a64 * 37) % N])
  dstf = jnp.concatenate([dst, N + (a64 % (N_PAD - N))])
  src64 = jnp.stack([srcf, srcf + N]).reshape(NC, NS, NBLK64, BLK)
  dst64 = dstf.reshape(NS, NBLK64, BLK)

  zeros16 = jnp.zeros((STRIPE, 16), jnp.float32)
  zeros64 = jnp.zeros((STRIPE, 64), jnp.float32)
  ones_blk = jnp.ones((BLK, 16), jnp.float32)

  dp = _deg16(ones_blk, dst16, zeros16)
  dinv, t1 = _tc_first(dp[0, :N, 0:1], dp[1, :N, 0:1], x, W1)

  p = _agg64(t1.reshape(NC * N, 64), src64, dst64, zeros64)
  t2 = _tc_mid(t1, p[0, :N], p[1, :N], dinv, b1.reshape(1, 128), W2, True)

  p = _agg64(t2.reshape(NC * N, 64), src64, dst64, zeros64)
  w3p = jnp.pad(W3, ((0, 0), (0, 16 - W3.shape[1])))
  y3 = _tc_mid(t2, p[0, :N], p[1, :N], dinv, b2.reshape(1, 128), w3p, False)

  r = _agg16(y3, src16, dst16, zeros16)
  b3p = jnp.pad(b3, (0, 16 - b3.shape[0])).reshape(1, 16)
  out16 = _tc_last(y3, r[0, :N], r[1, :N], dinv, b3p)
  return out16[:, :2]


# speedup vs baseline: 37.9097x; 1.5621x over previous
import functools

import jax
import jax.numpy as jnp
from jax import lax
from jax.experimental import pallas as pl
from jax.experimental.pallas import tpu as pltpu
from jax.experimental.pallas import tpu_sc as plsc

N = 10000
E = 640000
NC = 2
NS = 16
NW = NC * NS
BLK = 128
N_PAD = 10112
STRIPE = N_PAD // NS

NBLK16 = 160
E_PAD16 = NW * NBLK16 * BLK
WBLK = 160
NWIN = 2
NBLK64 = NWIN * WBLK
E_PAD64 = NS * NBLK64 * BLK

R = 2000

_SC_PARAMS = pltpu.CompilerParams(use_tc_tiling_on_sc=False)


def _agg_loop(nblk, tbl_hbm, src_v, dst_v, bufs, acc, gsems, ssems):
  nb = len(bufs)
  for b in range(nb - 1):
    pltpu.async_copy(tbl_hbm.at[src_v.at[b]], bufs[b], gsems[b])

  def body(i, carry):
    j0 = nb * i
    for b in range(nb):
      j = j0 + b
      pltpu.make_async_copy(tbl_hbm.at[src_v.at[j]], bufs[b],
                            gsems[b]).wait()
      pltpu.async_copy(bufs[b], acc.at[dst_v.at[j]], ssems[b], add=True)
      bn = (b + nb - 1) % nb

      @pl.when(j + nb - 1 < nblk)
      def _():
        @pl.when(j >= 1)
        def _():
          pltpu.make_async_copy(bufs[bn], acc.at[dst_v.at[0]],
                                ssems[bn]).wait()

        pltpu.async_copy(tbl_hbm.at[src_v.at[j + nb - 1]], bufs[bn],
                         gsems[bn])
    return carry

  lax.fori_loop(0, nblk // nb, body, 0)
  for b in range(nb):
    pltpu.make_async_copy(bufs[b], acc.at[dst_v.at[0]], ssems[b]).wait()


def _make_agg16():
  mesh = plsc.VectorSubcoreMesh(core_axis_name="c", subcore_axis_name="s")
  out_type = jax.ShapeDtypeStruct((NC, N_PAD, 16), jnp.float32)
  scratch = [
      pltpu.VMEM((NBLK16, BLK), jnp.int32),
      pltpu.VMEM((NBLK16, BLK), jnp.int32),
      [pltpu.VMEM((BLK, 16), jnp.float32)] * 4,
      pltpu.VMEM_SHARED((N_PAD, 16), jnp.float32),
      [pltpu.SemaphoreType.DMA] * 4,
      [pltpu.SemaphoreType.DMA] * 4,
  ]

  @functools.partial(pl.kernel, out_type=out_type, mesh=mesh,
                     scratch_types=scratch, compiler_params=_SC_PARAMS)
  def agg(tbl_hbm, src_hbm, dst_hbm, zeros_hbm, out_hbm,
          src_v, dst_v, bufs, acc, gsems, ssems):
    cid = lax.axis_index("c")
    sid = lax.axis_index("s")
    wid = sid * NC + cid
    pltpu.sync_copy(zeros_hbm, acc.at[pl.ds(sid * STRIPE, STRIPE)])
    pltpu.sync_copy(src_hbm.at[wid], src_v)
    pltpu.sync_copy(dst_hbm.at[wid], dst_v)
    plsc.subcore_barrier()
    _agg_loop(NBLK16, tbl_hbm, src_v, dst_v, bufs, acc, gsems, ssems)
    plsc.subcore_barrier()
    pltpu.sync_copy(acc.at[pl.ds(sid * STRIPE, STRIPE)],
                    out_hbm.at[cid, pl.ds(sid * STRIPE, STRIPE)])

  return agg


def _make_agg64():
  mesh = plsc.VectorSubcoreMesh(core_axis_name="c", subcore_axis_name="s")
  out_type = jax.ShapeDtypeStruct((NC, N_PAD, 64), jnp.float32)
  scratch = [
      pltpu.VMEM((WBLK, BLK), jnp.int32),
      pltpu.VMEM((WBLK, BLK), jnp.int32),
      [pltpu.VMEM((BLK, 64), jnp.float32)] * 4,
      pltpu.VMEM_SHARED((N_PAD, 64), jnp.float32),
      [pltpu.SemaphoreType.DMA] * 4,
      [pltpu.SemaphoreType.DMA] * 4,
  ]

  @functools.partial(pl.kernel, out_type=out_type, mesh=mesh,
                     scratch_types=scratch, compiler_params=_SC_PARAMS)
  def agg(tbl_hbm, src_hbm, dst_hbm, zeros_hbm, out_hbm,
          src_v, dst_v, bufs, acc, gsems, ssems):
    cid = lax.axis_index("c")
    sid = lax.axis_index("s")
    pltpu.sync_copy(zeros_hbm, acc.at[pl.ds(sid * STRIPE, STRIPE)])
    plsc.subcore_barrier()

    def win(w, carry):
      pltpu.sync_copy(src_hbm.at[cid, sid, pl.ds(w * WBLK, WBLK)], src_v)
      pltpu.sync_copy(dst_hbm.at[sid, pl.ds(w * WBLK, WBLK)], dst_v)
      _agg_loop(WBLK, tbl_hbm, src_v, dst_v, bufs, acc, gsems, ssems)
      return carry

    lax.fori_loop(0, NWIN, win, 0)
    plsc.subcore_barrier()
    pltpu.sync_copy(acc.at[pl.ds(sid * STRIPE, STRIPE)],
                    out_hbm.at[cid, pl.ds(sid * STRIPE, STRIPE)])

  return agg


def _make_deg16():
  mesh = plsc.VectorSubcoreMesh(core_axis_name="c", subcore_axis_name="s")
  out_type = jax.ShapeDtypeStruct((NC, N_PAD, 16), jnp.float32)
  scratch = [
      pltpu.VMEM((NBLK16, BLK), jnp.int32),
      pltpu.VMEM((BLK, 16), jnp.float32),
      pltpu.VMEM_SHARED((N_PAD, 16), jnp.float32),
      [pltpu.SemaphoreType.DMA] * 4,
  ]

  @functools.partial(pl.kernel, out_type=out_type, mesh=mesh,
                     scratch_types=scratch, compiler_params=_SC_PARAMS)
  def deg(ones_hbm, dst_hbm, zeros_hbm, out_hbm, dst_v, ones_v, acc, ssems):
    cid = lax.axis_index("c")
    sid = lax.axis_index("s")
    wid = sid * NC + cid
    pltpu.sync_copy(zeros_hbm, acc.at[pl.ds(sid * STRIPE, STRIPE)])
    pltpu.sync_copy(dst_hbm.at[wid], dst_v)
    pltpu.sync_copy(ones_hbm, ones_v)
    plsc.subcore_barrier()

    def body(i, carry):
      j0 = 4 * i
      for b in range(4):
        j = j0 + b

        @pl.when(j >= 4)
        def _():
          pltpu.make_async_copy(ones_v, acc.at[dst_v.at[0]],
                                ssems[b]).wait()

        pltpu.async_copy(ones_v, acc.at[dst_v.at[j]], ssems[b], add=True)
      return carry

    lax.fori_loop(0, NBLK16 // 4, body, 0)
    for b in range(4):
      pltpu.make_async_copy(ones_v, acc.at[dst_v.at[0]], ssems[b]).wait()
    plsc.subcore_barrier()
    pltpu.sync_copy(acc.at[pl.ds(sid * STRIPE, STRIPE)],
                    out_hbm.at[cid, pl.ds(sid * STRIPE, STRIPE)])

  return deg


_agg16 = _make_agg16()
_agg64 = _make_agg64()
_deg16 = _make_deg16()


def _row_spec(w):
  return pl.BlockSpec((R, w), lambda i: (i, 0))


def _full_spec(shape):
  return pl.BlockSpec(shape, lambda i: tuple(0 for _ in shape))


def _tc_first(p0, p1, x, w1):
  k_in = x.shape[1]

  def body(p0_ref, p1_ref, x_ref, w_ref, dinv_ref, t_ref):
    deg = 1.0 + p0_ref[...] + p1_ref[...]
    dinv = lax.rsqrt(deg)
    dinv_ref[...] = dinv
    y = dinv * jnp.dot(x_ref[...], w_ref[...],
                       preferred_element_type=jnp.float32,
                       precision=lax.Precision.HIGHEST)
    t_ref[0] = y[:, :64]
    t_ref[1] = y[:, 64:]

  return pl.pallas_call(
      body,
      grid=(N // R,),
      in_specs=[_row_spec(1), _row_spec(1), _row_spec(k_in),
                _full_spec((k_in, 128))],
      out_specs=[_row_spec(1), pl.BlockSpec((2, R, 64), lambda i: (0, i, 0))],
      out_shape=[jax.ShapeDtypeStruct((N, 1), jnp.float32),
                 jax.ShapeDtypeStruct((2, N, 64), jnp.float32)],
  )(p0, p1, x, w1)


def _tc_mid(t, q0, q1, dinv, b, w_next, split_out):
  w_out = w_next.shape[1]

  def body(t_ref, q0_ref, q1_ref, dinv_ref, b_ref, w_ref, o_ref):
    dinv = dinv_ref[...]
    s = jnp.concatenate(
        [t_ref[0] + q0_ref[...], t_ref[1] + q1_ref[...]], axis=1)
    h = jnp.maximum(dinv * s + b_ref[...], 0.0)
    o = dinv * jnp.dot(h, w_ref[...],
                       preferred_element_type=jnp.float32,
                       precision=lax.Precision.HIGHEST)
    if split_out:
      o_ref[0] = o[:, :64]
      o_ref[1] = o[:, 64:]
    else:
      o_ref[...] = o

  if split_out:
    out_spec = pl.BlockSpec((2, R, 64), lambda i: (0, i, 0))
    out_shape = jax.ShapeDtypeStruct((2, N, 64), jnp.float32)
  else:
    out_spec = _row_spec(w_out)
    out_shape = jax.ShapeDtypeStruct((N, w_out), jnp.float32)

  return pl.pallas_call(
      body,
      grid=(N // R,),
      in_specs=[pl.BlockSpec((2, R, 64), lambda i: (0, i, 0)),
                _row_spec(64), _row_spec(64), _row_spec(1),
                _full_spec((1, 128)), _full_spec((128, w_out))],
      out_specs=out_spec,
      out_shape=out_shape,
  )(t, q0, q1, dinv, b, w_next)


def _tc_last(y, r0, r1, dinv, b):
  def body(y_ref, r0_ref, r1_ref, dinv_ref, b_ref, o_ref):
    o_ref[...] = (dinv_ref[...] * (y_ref[...] + r0_ref[...] + r1_ref[...])
                  + b_ref[...])

  return pl.pallas_call(
      body,
      grid=(N // R,),
      in_specs=[_row_spec(16), _row_spec(16), _row_spec(16), _row_spec(1),
                _full_spec((1, 16))],
      out_specs=_row_spec(16),
      out_shape=jax.ShapeDtypeStruct((N, 16), jnp.float32),
  )(y, r0, r1, dinv, b)


def kernel(x, edge_index, W1, b1, W2, b2, W3, b3):
  src = edge_index[0]
  dst = edge_index[1]

  p16 = E_PAD16 - E
  a16 = jnp.arange(p16, dtype=jnp.int32)
  src16 = jnp.concatenate([src, (a16 * 37) % N]).reshape(NW, NBLK16, BLK)
  dst16 = jnp.concatenate([dst, N + (a16 % (N_PAD - N))]).reshape(
      NW, NBLK16, BLK)

  p64 = E_PAD64 - E
  a64 = jnp.arange(p64, dtype=jnp.int32)
  srcf = jnp.concatenate([src, (a64 * 37) % N])
  dstf = jnp.concatenate([dst, N + (a64 % (N_PAD - N))])
  src64 = jnp.stack([srcf, srcf + N]).reshape(NC, NS, NBLK64, BLK)
  dst64 = dstf.reshape(NS, NBLK64, BLK)

  zeros16 = jnp.zeros((STRIPE, 16), jnp.float32)
  zeros64 = jnp.zeros((STRIPE, 64), jnp.float32)
  ones_blk = jnp.ones((BLK, 16), jnp.float32)

  dp = _deg16(ones_blk, dst16, zeros16)
  dinv, t1 = _tc_first(dp[0, :N, 0:1], dp[1, :N, 0:1], x, W1)

  p = _agg64(t1.reshape(NC * N, 64), src64, dst64, zeros64)
  t2 = _tc_mid(t1, p[0, :N], p[1, :N], dinv, b1.reshape(1, 128), W2, True)

  p = _agg64(t2.reshape(NC * N, 64), src64, dst64, zeros64)
  w3p = jnp.pad(W3, ((0, 0), (0, 16 - W3.shape[1])))
  y3 = _tc_mid(t2, p[0, :N], p[1, :N], dinv, b2.reshape(1, 128), w3p, False)

  r = _agg16(y3, src16, dst16, zeros16)
  b3p = jnp.pad(b3, (0, 16 - b3.shape[0])).reshape(1, 16)
  out16 = _tc_last(y3, r[0, :N], r[1, :N], dinv, b3p)
  return out16[:, :2]


# speedup vs baseline: 44.7203x; 1.1797x over previous
import functools

import jax
import jax.numpy as jnp
from jax import lax
from jax.experimental import pallas as pl
from jax.experimental.pallas import tpu as pltpu
from jax.experimental.pallas import tpu_sc as plsc

N = 10000
E = 640000
NC = 2
NS = 16
NW = NC * NS
BLK = 128
N_PAD = 10112
STRIPE = N_PAD // NS

NBLK16 = 160
E_PAD16 = NW * NBLK16 * BLK
WBLK = 160
NWIN = 2
NBLK64 = NWIN * WBLK
E_PAD64 = NS * NBLK64 * BLK

R = 2000

_SC_PARAMS = pltpu.CompilerParams(use_tc_tiling_on_sc=False)


def _agg_loop(nblk, tbl, src_v, dst_v, bufs, acc, gsems, ssems):
  nb = len(bufs)

  def gref(j, b):
    return tbl.at[src_v.at[j]], bufs[b], gsems[b]

  for b in range(nb - 1):
    pltpu.async_copy(*gref(b, b))

  def body(i, carry):
    j0 = nb * i
    for b in range(nb):
      j = j0 + b
      pltpu.make_async_copy(*gref(j, b)).wait()
      pltpu.async_copy(bufs[b], acc.at[dst_v.at[j]], ssems[b], add=True)
      bn = (b + nb - 1) % nb

      @pl.when(j + nb - 1 < nblk)
      def _():
        @pl.when(j >= 1)
        def _():
          pltpu.make_async_copy(bufs[bn], acc.at[dst_v.at[0]],
                                ssems[bn]).wait()

        pltpu.async_copy(*gref(j + nb - 1, bn))
    return carry

  lax.fori_loop(0, nblk // nb, body, 0)
  for b in range(nb):
    pltpu.make_async_copy(bufs[b], acc.at[dst_v.at[0]], ssems[b]).wait()


def _make_agg64():
  mesh = plsc.VectorSubcoreMesh(core_axis_name="c", subcore_axis_name="s")
  out_type = jax.ShapeDtypeStruct((N_PAD, 128), jnp.float32)
  scratch = [
      pltpu.VMEM((WBLK, BLK), jnp.int32),
      pltpu.VMEM((WBLK, BLK), jnp.int32),
      [pltpu.VMEM((BLK, 64), jnp.float32)] * 4,
      pltpu.VMEM_SHARED((N_PAD, 64), jnp.float32),
      [pltpu.SemaphoreType.DMA] * 4,
      [pltpu.SemaphoreType.DMA] * 4,
  ]

  @functools.partial(pl.kernel, out_type=out_type, mesh=mesh,
                     scratch_types=scratch, compiler_params=_SC_PARAMS)
  def agg(tbl_hbm, src_hbm, dst_hbm, zeros_hbm, out_hbm,
          src_v, dst_v, bufs, acc, gsems, ssems):
    cid = lax.axis_index("c")
    sid = lax.axis_index("s")
    rows = pl.ds(sid * STRIPE, STRIPE)
    pltpu.sync_copy(zeros_hbm.at[rows, pl.ds(0, 64)], acc.at[rows])
    plsc.subcore_barrier()

    def win(w, carry):
      pltpu.sync_copy(src_hbm.at[cid, sid, pl.ds(w * WBLK, WBLK)], src_v)
      pltpu.sync_copy(dst_hbm.at[sid, pl.ds(w * WBLK, WBLK)], dst_v)
      _agg_loop(WBLK, tbl_hbm, src_v, dst_v, bufs, acc, gsems, ssems)
      return carry

    lax.fori_loop(0, NWIN, win, 0)
    plsc.subcore_barrier()
    pltpu.sync_copy(acc.at[rows], out_hbm.at[rows, pl.ds(cid * 64, 64)])

  return agg


def _make_agg16():
  mesh = plsc.VectorSubcoreMesh(core_axis_name="c", subcore_axis_name="s")
  out_type = jax.ShapeDtypeStruct((N_PAD, 128), jnp.float32)
  scratch = [
      pltpu.VMEM((NBLK16, BLK), jnp.int32),
      pltpu.VMEM((NBLK16, BLK), jnp.int32),
      [pltpu.VMEM((BLK, 16), jnp.float32)] * 4,
      pltpu.VMEM_SHARED((N_PAD, 16), jnp.float32),
      [pltpu.SemaphoreType.DMA] * 4,
      [pltpu.SemaphoreType.DMA] * 4,
  ]

  @functools.partial(pl.kernel, out_type=out_type, mesh=mesh,
                     scratch_types=scratch, compiler_params=_SC_PARAMS)
  def agg(tbl_hbm, src_hbm, dst_hbm, zeros_hbm, out_hbm,
          src_v, dst_v, bufs, acc, gsems, ssems):
    cid = lax.axis_index("c")
    sid = lax.axis_index("s")
    wid = sid * NC + cid
    rows = pl.ds(sid * STRIPE, STRIPE)
    pltpu.sync_copy(zeros_hbm.at[rows, pl.ds(0, 16)], acc.at[rows])
    pltpu.sync_copy(src_hbm.at[wid], src_v)
    pltpu.sync_copy(dst_hbm.at[wid], dst_v)
    plsc.subcore_barrier()
    _agg_loop(NBLK16, tbl_hbm, src_v, dst_v, bufs, acc, gsems, ssems)
    plsc.subcore_barrier()
    pltpu.sync_copy(acc.at[rows], out_hbm.at[rows, pl.ds(cid * 16, 16)])

  return agg


def _make_deg16():
  mesh = plsc.VectorSubcoreMesh(core_axis_name="c", subcore_axis_name="s")
  out_type = jax.ShapeDtypeStruct((N_PAD, 128), jnp.float32)
  scratch = [
      pltpu.VMEM((NBLK16, BLK), jnp.int32),
      pltpu.VMEM((BLK, 16), jnp.float32),
      pltpu.VMEM_SHARED((N_PAD, 16), jnp.float32),
      [pltpu.SemaphoreType.DMA] * 4,
  ]

  @functools.partial(pl.kernel, out_type=out_type, mesh=mesh,
                     scratch_types=scratch, compiler_params=_SC_PARAMS)
  def deg(ones_hbm, dst_hbm, zeros_hbm, out_hbm, dst_v, ones_v, acc, ssems):
    cid = lax.axis_index("c")
    sid = lax.axis_index("s")
    wid = sid * NC + cid
    rows = pl.ds(sid * STRIPE, STRIPE)
    pltpu.sync_copy(zeros_hbm.at[rows, pl.ds(0, 16)], acc.at[rows])
    pltpu.sync_copy(dst_hbm.at[wid], dst_v)
    pltpu.sync_copy(ones_hbm, ones_v)
    plsc.subcore_barrier()

    def body(i, carry):
      j0 = 4 * i
      for b in range(4):
        j = j0 + b

        @pl.when(j >= 4)
        def _():
          pltpu.make_async_copy(ones_v, acc.at[dst_v.at[0]],
                                ssems[b]).wait()

        pltpu.async_copy(ones_v, acc.at[dst_v.at[j]], ssems[b], add=True)
      return carry

    lax.fori_loop(0, NBLK16 // 4, body, 0)
    for b in range(4):
      pltpu.make_async_copy(ones_v, acc.at[dst_v.at[0]], ssems[b]).wait()
    plsc.subcore_barrier()
    pltpu.sync_copy(acc.at[rows], out_hbm.at[rows, pl.ds(cid * 16, 16)])

  return deg


_agg16 = _make_agg16()
_agg64 = _make_agg64()
_deg16 = _make_deg16()


def _row_spec(w):
  return pl.BlockSpec((R, w), lambda i: (i, 0))


def _full_spec(shape):
  return pl.BlockSpec(shape, lambda i: tuple(0 for _ in shape))


def _tc_first(dp, x, w1):
  k_in = x.shape[1]

  def body(dp_ref, x_ref, w_ref, dinv_ref, y_ref):
    deg = 1.0 + dp_ref[:, 0:1] + dp_ref[:, 16:17]
    dinv = lax.rsqrt(deg)
    dinv_ref[...] = dinv
    y_ref[...] = dinv * jnp.dot(x_ref[...], w_ref[...],
                                preferred_element_type=jnp.float32,
                                precision=lax.Precision.HIGHEST)

  return pl.pallas_call(
      body,
      grid=(N // R,),
      in_specs=[_row_spec(128), _row_spec(k_in), _full_spec((k_in, 128))],
      out_specs=[_row_spec(1), _row_spec(128)],
      out_shape=[jax.ShapeDtypeStruct((N, 1), jnp.float32),
                 jax.ShapeDtypeStruct((N, 128), jnp.float32)],
  )(dp, x, w1)


def _tc_mid(y, q, dinv, b, w_next):
  def body(y_ref, q_ref, dinv_ref, b_ref, w_ref, o_ref):
    dinv = dinv_ref[...]
    h = jnp.maximum(dinv * (y_ref[...] + q_ref[...]) + b_ref[...], 0.0)
    o_ref[...] = dinv * jnp.dot(h, w_ref[...],
                                preferred_element_type=jnp.float32,
                                precision=lax.Precision.HIGHEST)

  return pl.pallas_call(
      body,
      grid=(N // R,),
      in_specs=[_row_spec(128), _row_spec(128), _row_spec(1),
                _full_spec((1, 128)), _full_spec((128, 128))],
      out_specs=_row_spec(128),
      out_shape=jax.ShapeDtypeStruct((N, 128), jnp.float32),
  )(y, q, dinv, b, w_next)


def _tc_last(y, r, dinv, b):
  def body(y_ref, r_ref, dinv_ref, b_ref, o_ref):
    s = y_ref[:, 0:16] + r_ref[:, 0:16] + r_ref[:, 16:32]
    o_ref[...] = dinv_ref[...] * s + b_ref[...]

  return pl.pallas_call(
      body,
      grid=(N // R,),
      in_specs=[_row_spec(128), _row_spec(128), _row_spec(1),
                _full_spec((1, 16))],
      out_specs=_row_spec(16),
      out_shape=jax.ShapeDtypeStruct((N, 16), jnp.float32),
  )(y, r, dinv, b)


def kernel(x, edge_index, W1, b1, W2, b2, W3, b3):
  src = edge_index[0]
  dst = edge_index[1]

  p16 = E_PAD16 - E
  a16 = jnp.arange(p16, dtype=jnp.int32)
  src16 = (8 * jnp.concatenate([src, (a16 * 37) % N])).reshape(
      NW, NBLK16, BLK)
  dst16 = jnp.concatenate([dst, N + (a16 % (N_PAD - N))]).reshape(
      NW, NBLK16, BLK)

  p64 = E_PAD64 - E
  a64 = jnp.arange(p64, dtype=jnp.int32)
  srcf = 2 * jnp.concatenate([src, (a64 * 37) % N])
  src64 = jnp.stack([srcf, srcf + 1]).reshape(NC, NS, NBLK64, BLK)
  dst64 = jnp.concatenate([dst, N + (a64 % (N_PAD - N))]).reshape(
      NS, NBLK64, BLK)

  zeros = jnp.zeros((N_PAD, 64), jnp.float32)
  ones_blk = jnp.ones((BLK, 16), jnp.float32)

  dp = _deg16(ones_blk, dst16, zeros)
  dinv, y1 = _tc_first(dp, x, W1)

  q = _agg64(y1.reshape(2 * N, 64), src64, dst64, zeros)
  y2 = _tc_mid(y1, q, dinv, b1.reshape(1, 128), W2)

  q = _agg64(y2.reshape(2 * N, 64), src64, dst64, zeros)
  w3p = jnp.pad(W3, ((0, 0), (0, 128 - W3.shape[1])))
  y3 = _tc_mid(y2, q, dinv, b2.reshape(1, 128), w3p)

  r = _agg16(y3.reshape(8 * N, 16), src16, dst16, zeros)
  b3p = jnp.pad(b3, (0, 16 - b3.shape[0])).reshape(1, 16)
  out16 = _tc_last(y3, r, dinv, b3p)
  return out16[:, :2]


# speedup vs baseline: 46.2093x; 1.0333x over previous
import functools

import jax
import jax.numpy as jnp
from jax import lax
from jax.experimental import pallas as pl
from jax.experimental.pallas import tpu as pltpu
from jax.experimental.pallas import tpu_sc as plsc

N = 10000
E = 640000
NC = 2
NS = 16
NW = NC * NS
BLK = 128
N_PAD = 10112
STRIPE = N_PAD // NS

NBLK16 = 160
E_PAD16 = NW * NBLK16 * BLK
WBLK = 160
NWIN = 2
NBLK64 = NWIN * WBLK
E_PAD64 = NS * NBLK64 * BLK

R = 2000

_SC_PARAMS = pltpu.CompilerParams(use_tc_tiling_on_sc=False)


def _agg_loop(nblk, tbl, src_v, dst_v, bufs, acc, gsems, ssems):
  nb = len(bufs)

  def gref(j, b):
    return tbl.at[src_v.at[j]], bufs[b], gsems[b]

  for b in range(nb - 1):
    pltpu.async_copy(*gref(b, b))

  def body(i, carry):
    j0 = nb * i
    for b in range(nb):
      j = j0 + b
      pltpu.make_async_copy(*gref(j, b)).wait()
      pltpu.async_copy(bufs[b], acc.at[dst_v.at[j]], ssems[b], add=True)
      bn = (b + nb - 1) % nb

      @pl.when(j + nb - 1 < nblk)
      def _():
        @pl.when(j >= 1)
        def _():
          pltpu.make_async_copy(bufs[bn], acc.at[dst_v.at[0]],
                                ssems[bn]).wait()

        pltpu.async_copy(*gref(j + nb - 1, bn))
    return carry

  lax.fori_loop(0, nblk // nb, body, 0)
  for b in range(nb):
    pltpu.make_async_copy(bufs[b], acc.at[dst_v.at[0]], ssems[b]).wait()


def _make_agg64():
  mesh = plsc.VectorSubcoreMesh(core_axis_name="c", subcore_axis_name="s")
  out_type = jax.ShapeDtypeStruct((N_PAD, 128), jnp.float32)
  scratch = [
      pltpu.VMEM((WBLK, BLK), jnp.int32),
      pltpu.VMEM((WBLK, BLK), jnp.int32),
      [pltpu.VMEM((BLK, 64), jnp.float32)] * 4,
      pltpu.VMEM_SHARED((N_PAD, 64), jnp.float32),
      [pltpu.SemaphoreType.DMA] * 4,
      [pltpu.SemaphoreType.DMA] * 4,
  ]

  @functools.partial(pl.kernel, out_type=out_type, mesh=mesh,
                     scratch_types=scratch, compiler_params=_SC_PARAMS)
  def agg(tbl_hbm, src_hbm, dst_hbm, zeros_hbm, out_hbm,
          src_v, dst_v, bufs, acc, gsems, ssems):
    cid = lax.axis_index("c")
    sid = lax.axis_index("s")
    rows = pl.ds(sid * STRIPE, STRIPE)
    pltpu.sync_copy(zeros_hbm.at[rows, pl.ds(0, 64)], acc.at[rows])
    plsc.subcore_barrier()

    def win(w, carry):
      pltpu.sync_copy(src_hbm.at[cid, sid, pl.ds(w * WBLK, WBLK)], src_v)
      pltpu.sync_copy(dst_hbm.at[sid, pl.ds(w * WBLK, WBLK)], dst_v)
      _agg_loop(WBLK, tbl_hbm, src_v, dst_v, bufs, acc, gsems, ssems)
      return carry

    lax.fori_loop(0, NWIN, win, 0)
    plsc.subcore_barrier()
    pltpu.sync_copy(acc.at[rows], out_hbm.at[rows, pl.ds(cid * 64, 64)])

  return agg


def _make_agg16():
  mesh = plsc.VectorSubcoreMesh(core_axis_name="c", subcore_axis_name="s")
  out_type = jax.ShapeDtypeStruct((N_PAD, 128), jnp.float32)
  scratch = [
      pltpu.VMEM((NBLK16, BLK), jnp.int32),
      pltpu.VMEM((NBLK16, BLK), jnp.int32),
      [pltpu.VMEM((BLK, 16), jnp.float32)] * 8,
      pltpu.VMEM_SHARED((N_PAD, 16), jnp.float32),
      [pltpu.SemaphoreType.DMA] * 8,
      [pltpu.SemaphoreType.DMA] * 8,
  ]

  @functools.partial(pl.kernel, out_type=out_type, mesh=mesh,
                     scratch_types=scratch, compiler_params=_SC_PARAMS)
  def agg(tbl_hbm, src_hbm, dst_hbm, zeros_hbm, out_hbm,
          src_v, dst_v, bufs, acc, gsems, ssems):
    cid = lax.axis_index("c")
    sid = lax.axis_index("s")
    wid = sid * NC + cid
    rows = pl.ds(sid * STRIPE, STRIPE)
    pltpu.sync_copy(zeros_hbm.at[rows, pl.ds(0, 16)], acc.at[rows])
    pltpu.sync_copy(src_hbm.at[wid], src_v)
    pltpu.sync_copy(dst_hbm.at[wid], dst_v)
    plsc.subcore_barrier()
    _agg_loop(NBLK16, tbl_hbm, src_v, dst_v, bufs, acc, gsems, ssems)
    plsc.subcore_barrier()
    pltpu.sync_copy(acc.at[rows], out_hbm.at[rows, pl.ds(cid * 16, 16)])

  return agg


def _make_deg16():
  mesh = plsc.VectorSubcoreMesh(core_axis_name="c", subcore_axis_name="s")
  out_type = jax.ShapeDtypeStruct((N_PAD, 128), jnp.float32)
  scratch = [
      pltpu.VMEM((NBLK16, BLK), jnp.int32),
      pltpu.VMEM((BLK, 16), jnp.float32),
      pltpu.VMEM_SHARED((N_PAD, 16), jnp.float32),
      [pltpu.SemaphoreType.DMA] * 4,
  ]

  @functools.partial(pl.kernel, out_type=out_type, mesh=mesh,
                     scratch_types=scratch, compiler_params=_SC_PARAMS)
  def deg(ones_hbm, dst_hbm, zeros_hbm, out_hbm, dst_v, ones_v, acc, ssems):
    cid = lax.axis_index("c")
    sid = lax.axis_index("s")
    wid = sid * NC + cid
    rows = pl.ds(sid * STRIPE, STRIPE)
    pltpu.sync_copy(zeros_hbm.at[rows, pl.ds(0, 16)], acc.at[rows])
    pltpu.sync_copy(dst_hbm.at[wid], dst_v)
    pltpu.sync_copy(ones_hbm, ones_v)
    plsc.subcore_barrier()

    def body(i, carry):
      j0 = 4 * i
      for b in range(4):
        j = j0 + b

        @pl.when(j >= 4)
        def _():
          pltpu.make_async_copy(ones_v, acc.at[dst_v.at[0]],
                                ssems[b]).wait()

        pltpu.async_copy(ones_v, acc.at[dst_v.at[j]], ssems[b], add=True)
      return carry

    lax.fori_loop(0, NBLK16 // 4, body, 0)
    for b in range(4):
      pltpu.make_async_copy(ones_v, acc.at[dst_v.at[0]], ssems[b]).wait()
    plsc.subcore_barrier()
    pltpu.sync_copy(acc.at[rows], out_hbm.at[rows, pl.ds(cid * 16, 16)])

  return deg


_agg16 = _make_agg16()
_agg64 = _make_agg64()
_deg16 = _make_deg16()


def _row_spec(w):
  return pl.BlockSpec((R, w), lambda i: (i, 0))


def _full_spec(shape):
  return pl.BlockSpec(shape, lambda i: tuple(0 for _ in shape))


def _tc_first(dp, x, w1):
  k_in = x.shape[1]

  def body(dp_ref, x_ref, w_ref, dinv_ref, y_ref):
    deg = 1.0 + dp_ref[:, 0:1] + dp_ref[:, 16:17]
    dinv = lax.rsqrt(deg)
    dinv_ref[...] = dinv
    y_ref[...] = dinv * jnp.dot(x_ref[...], w_ref[...],
                                preferred_element_type=jnp.float32,
                                precision=lax.Precision.HIGHEST)

  return pl.pallas_call(
      body,
      grid=(N // R,),
      in_specs=[_row_spec(128), _row_spec(k_in), _full_spec((k_in, 128))],
      out_specs=[_row_spec(1), _row_spec(128)],
      out_shape=[jax.ShapeDtypeStruct((N, 1), jnp.float32),
                 jax.ShapeDtypeStruct((N, 128), jnp.float32)],
  )(dp, x, w1)


def _tc_mid(y, q, dinv, b, w_next):
  def body(y_ref, q_ref, dinv_ref, b_ref, w_ref, o_ref):
    dinv = dinv_ref[...]
    h = jnp.maximum(dinv * (y_ref[...] + q_ref[...]) + b_ref[...], 0.0)
    o_ref[...] = dinv * jnp.dot(h, w_ref[...],
                                preferred_element_type=jnp.float32,
                                precision=lax.Precision.HIGHEST)

  return pl.pallas_call(
      body,
      grid=(N // R,),
      in_specs=[_row_spec(128), _row_spec(128), _row_spec(1),
                _full_spec((1, 128)), _full_spec((128, 128))],
      out_specs=_row_spec(128),
      out_shape=jax.ShapeDtypeStruct((N, 128), jnp.float32),
  )(y, q, dinv, b, w_next)


def _tc_last(y, r, dinv, b):
  def body(y_ref, r_ref, dinv_ref, b_ref, o_ref):
    s = y_ref[:, 0:2] + r_ref[:, 0:2] + r_ref[:, 16:18]
    o_ref[...] = dinv_ref[...] * s + b_ref[...]

  return pl.pallas_call(
      body,
      grid=(N // R,),
      in_specs=[_row_spec(128), _row_spec(128), _row_spec(1),
                _full_spec((1, 2))],
      out_specs=_row_spec(2),
      out_shape=jax.ShapeDtypeStruct((N, 2), jnp.float32),
  )(y, r, dinv, b)


def kernel(x, edge_index, W1, b1, W2, b2, W3, b3):
  src = edge_index[0]
  dst = edge_index[1]

  p16 = E_PAD16 - E
  a16 = jnp.arange(p16, dtype=jnp.int32)
  src16 = (8 * jnp.concatenate([src, (a16 * 37) % N])).reshape(
      NW, NBLK16, BLK)
  dst16 = jnp.concatenate([dst, N + (a16 % (N_PAD - N))]).reshape(
      NW, NBLK16, BLK)

  p64 = E_PAD64 - E
  a64 = jnp.arange(p64, dtype=jnp.int32)
  srcf = 2 * jnp.concatenate([src, (a64 * 37) % N])
  src64 = jnp.stack([srcf, srcf + 1]).reshape(NC, NS, NBLK64, BLK)
  dst64 = jnp.concatenate([dst, N + (a64 % (N_PAD - N))]).reshape(
      NS, NBLK64, BLK)

  zeros = jnp.zeros((N_PAD, 64), jnp.float32)
  ones_blk = jnp.ones((BLK, 16), jnp.float32)

  dp = _deg16(ones_blk, dst16, zeros)
  dinv, y1 = _tc_first(dp, x, W1)

  q = _agg64(y1.reshape(2 * N, 64), src64, dst64, zeros)
  y2 = _tc_mid(y1, q, dinv, b1.reshape(1, 128), W2)

  q = _agg64(y2.reshape(2 * N, 64), src64, dst64, zeros)
  w3p = jnp.pad(W3, ((0, 0), (0, 128 - W3.shape[1])))
  y3 = _tc_mid(y2, q, dinv, b2.reshape(1, 128), w3p)

  r = _agg16(y3.reshape(8 * N, 16), src16, dst16, zeros)
  return _tc_last(y3, r, dinv, b3.reshape(1, 2))


# speedup vs baseline: 46.6798x; 1.0102x over previous
import functools

import jax
import jax.numpy as jnp
from jax import lax
from jax.experimental import pallas as pl
from jax.experimental.pallas import tpu as pltpu
from jax.experimental.pallas import tpu_sc as plsc

N = 10000
E = 640000
NC = 2
NS = 16
NW = NC * NS
BLK = 128
N_PAD = 10112
STRIPE = N_PAD // NS

NBLK16 = 160
E_PAD16 = NW * NBLK16 * BLK
WBLK = 160
NWIN = 2
NBLK64 = NWIN * WBLK
E_PAD64 = NS * NBLK64 * BLK

R = 2000

_SC_PARAMS = pltpu.CompilerParams(use_tc_tiling_on_sc=False)


def _agg_loop(nblk, tbl, src_v, dst_v, bufs, acc, gsems, ssems):
  nb = len(bufs)

  def gref(j, b):
    return tbl.at[src_v.at[j]], bufs[b], gsems[b]

  for b in range(nb - 1):
    pltpu.async_copy(*gref(b, b))

  def body(i, carry):
    j0 = nb * i
    for b in range(nb):
      j = j0 + b
      pltpu.make_async_copy(*gref(j, b)).wait()
      pltpu.async_copy(bufs[b], acc.at[dst_v.at[j]], ssems[b], add=True)
      bn = (b + nb - 1) % nb

      @pl.when(j + nb - 1 < nblk)
      def _():
        @pl.when(j >= 1)
        def _():
          pltpu.make_async_copy(bufs[bn], acc.at[dst_v.at[0]],
                                ssems[bn]).wait()

        pltpu.async_copy(*gref(j + nb - 1, bn))
    return carry

  lax.fori_loop(0, nblk // nb, body, 0)
  for b in range(nb):
    pltpu.make_async_copy(bufs[b], acc.at[dst_v.at[0]], ssems[b]).wait()


def _make_agg64():
  mesh = plsc.VectorSubcoreMesh(core_axis_name="c", subcore_axis_name="s")
  out_type = jax.ShapeDtypeStruct((N_PAD, 128), jnp.float32)
  scratch = [
      pltpu.VMEM((WBLK, BLK), jnp.int32),
      pltpu.VMEM((WBLK, BLK), jnp.int32),
      [pltpu.VMEM((BLK, 64), jnp.float32)] * 4,
      pltpu.VMEM_SHARED((N_PAD, 64), jnp.float32),
      [pltpu.SemaphoreType.DMA] * 4,
      [pltpu.SemaphoreType.DMA] * 4,
  ]

  @functools.partial(pl.kernel, out_type=out_type, mesh=mesh,
                     scratch_types=scratch, compiler_params=_SC_PARAMS)
  def agg(tbl_hbm, src_hbm, dst_hbm, zeros_hbm, out_hbm,
          src_v, dst_v, bufs, acc, gsems, ssems):
    cid = lax.axis_index("c")
    sid = lax.axis_index("s")
    rows = pl.ds(sid * STRIPE, STRIPE)
    pltpu.sync_copy(zeros_hbm.at[rows, pl.ds(0, 64)], acc.at[rows])
    plsc.subcore_barrier()

    def win(w, carry):
      pltpu.sync_copy(src_hbm.at[cid, sid, pl.ds(w * WBLK, WBLK)], src_v)
      pltpu.sync_copy(dst_hbm.at[sid, pl.ds(w * WBLK, WBLK)], dst_v)
      _agg_loop(WBLK, tbl_hbm, src_v, dst_v, bufs, acc, gsems, ssems)
      return carry

    lax.fori_loop(0, NWIN, win, 0)
    plsc.subcore_barrier()
    pltpu.sync_copy(acc.at[rows], out_hbm.at[rows, pl.ds(cid * 64, 64)])

  return agg


def _make_agg16():
  mesh = plsc.VectorSubcoreMesh(core_axis_name="c", subcore_axis_name="s")
  out_type = jax.ShapeDtypeStruct((N_PAD, 128), jnp.float32)
  scratch = [
      pltpu.VMEM((NBLK16, BLK), jnp.int32),
      pltpu.VMEM((NBLK16, BLK), jnp.int32),
      [pltpu.VMEM((BLK, 16), jnp.float32)] * 8,
      pltpu.VMEM_SHARED((N_PAD, 16), jnp.float32),
      [pltpu.SemaphoreType.DMA] * 8,
      [pltpu.SemaphoreType.DMA] * 8,
  ]

  @functools.partial(pl.kernel, out_type=out_type, mesh=mesh,
                     scratch_types=scratch, compiler_params=_SC_PARAMS)
  def agg(tbl_hbm, src_hbm, dst_hbm, zeros_hbm, out_hbm,
          src_v, dst_v, bufs, acc, gsems, ssems):
    cid = lax.axis_index("c")
    sid = lax.axis_index("s")
    wid = sid * NC + cid
    rows = pl.ds(sid * STRIPE, STRIPE)
    pltpu.sync_copy(zeros_hbm.at[rows, pl.ds(0, 16)], acc.at[rows])
    pltpu.sync_copy(src_hbm.at[wid], src_v)
    pltpu.sync_copy(dst_hbm.at[wid], dst_v)
    plsc.subcore_barrier()
    _agg_loop(NBLK16, tbl_hbm, src_v, dst_v, bufs, acc, gsems, ssems)
    plsc.subcore_barrier()
    pltpu.sync_copy(acc.at[rows], out_hbm.at[rows, pl.ds(cid * 16, 16)])

  return agg


def _make_deg16():
  mesh = plsc.VectorSubcoreMesh(core_axis_name="c", subcore_axis_name="s")
  out_type = jax.ShapeDtypeStruct((N_PAD, 128), jnp.float32)
  scratch = [
      pltpu.VMEM((NBLK16, BLK), jnp.int32),
      pltpu.VMEM((BLK, 16), jnp.float32),
      pltpu.VMEM_SHARED((N_PAD, 16), jnp.float32),
      [pltpu.SemaphoreType.DMA] * 4,
  ]

  @functools.partial(pl.kernel, out_type=out_type, mesh=mesh,
                     scratch_types=scratch, compiler_params=_SC_PARAMS)
  def deg(ones_hbm, dst_hbm, zeros_hbm, out_hbm, dst_v, ones_v, acc, ssems):
    cid = lax.axis_index("c")
    sid = lax.axis_index("s")
    wid = sid * NC + cid
    rows = pl.ds(sid * STRIPE, STRIPE)
    pltpu.sync_copy(zeros_hbm.at[rows, pl.ds(0, 16)], acc.at[rows])
    pltpu.sync_copy(dst_hbm.at[wid], dst_v)
    pltpu.sync_copy(ones_hbm, ones_v)
    plsc.subcore_barrier()

    def body(i, carry):
      j0 = 4 * i
      for b in range(4):
        j = j0 + b

        @pl.when(j >= 4)
        def _():
          pltpu.make_async_copy(ones_v, acc.at[dst_v.at[0]],
                                ssems[b]).wait()

        pltpu.async_copy(ones_v, acc.at[dst_v.at[j]], ssems[b], add=True)
      return carry

    lax.fori_loop(0, NBLK16 // 4, body, 0)
    for b in range(4):
      pltpu.make_async_copy(ones_v, acc.at[dst_v.at[0]], ssems[b]).wait()
    plsc.subcore_barrier()
    pltpu.sync_copy(acc.at[rows], out_hbm.at[rows, pl.ds(cid * 16, 16)])

  return deg


_agg16 = _make_agg16()
_agg64 = _make_agg64()
_deg16 = _make_deg16()


def _row_spec(w):
  return pl.BlockSpec((R, w), lambda i: (i, 0))


def _full_spec(shape):
  return pl.BlockSpec(shape, lambda i: tuple(0 for _ in shape))


def _tc_first(dp, x, w1):
  k_in = x.shape[1]

  def body(dp_ref, x_ref, w_ref, dinv_ref, y_ref):
    deg = 1.0 + dp_ref[:, 0:1] + dp_ref[:, 16:17]
    dinv = lax.rsqrt(deg)
    dinv_ref[...] = dinv
    y_ref[...] = dinv * jnp.dot(x_ref[...], w_ref[...],
                                preferred_element_type=jnp.float32,
                                precision=lax.Precision.DEFAULT)

  return pl.pallas_call(
      body,
      grid=(N // R,),
      in_specs=[_row_spec(128), _row_spec(k_in), _full_spec((k_in, 128))],
      out_specs=[_row_spec(1), _row_spec(128)],
      out_shape=[jax.ShapeDtypeStruct((N, 1), jnp.float32),
                 jax.ShapeDtypeStruct((N, 128), jnp.float32)],
  )(dp, x, w1)


def _tc_mid(y, q, dinv, b, w_next):
  def body(y_ref, q_ref, dinv_ref, b_ref, w_ref, o_ref):
    dinv = dinv_ref[...]
    h = jnp.maximum(dinv * (y_ref[...] + q_ref[...]) + b_ref[...], 0.0)
    o_ref[...] = dinv * jnp.dot(h, w_ref[...],
                                preferred_element_type=jnp.float32,
                                precision=lax.Precision.DEFAULT)

  return pl.pallas_call(
      body,
      grid=(N // R,),
      in_specs=[_row_spec(128), _row_spec(128), _row_spec(1),
                _full_spec((1, 128)), _full_spec((128, 128))],
      out_specs=_row_spec(128),
      out_shape=jax.ShapeDtypeStruct((N, 128), jnp.float32),
  )(y, q, dinv, b, w_next)


def _tc_last(y, r, dinv, b):
  def body(y_ref, r_ref, dinv_ref, b_ref, o_ref):
    s = y_ref[:, 0:2] + r_ref[:, 0:2] + r_ref[:, 16:18]
    o_ref[...] = dinv_ref[...] * s + b_ref[...]

  return pl.pallas_call(
      body,
      grid=(N // R,),
      in_specs=[_row_spec(128), _row_spec(128), _row_spec(1),
                _full_spec((1, 2))],
      out_specs=_row_spec(2),
      out_shape=jax.ShapeDtypeStruct((N, 2), jnp.float32),
  )(y, r, dinv, b)


def kernel(x, edge_index, W1, b1, W2, b2, W3, b3):
  src = edge_index[0]
  dst = edge_index[1]

  p16 = E_PAD16 - E
  a16 = jnp.arange(p16, dtype=jnp.int32)
  src16 = (8 * jnp.concatenate([src, (a16 * 37) % N])).reshape(
      NW, NBLK16, BLK)
  dst16 = jnp.concatenate([dst, N + (a16 % (N_PAD - N))]).reshape(
      NW, NBLK16, BLK)

  p64 = E_PAD64 - E
  a64 = jnp.arange(p64, dtype=jnp.int32)
  srcf = 2 * jnp.concatenate([src, (a64 * 37) % N])
  src64 = jnp.stack([srcf, srcf + 1]).reshape(NC, NS, NBLK64, BLK)
  dst64 = jnp.concatenate([dst, N + (a64 % (N_PAD - N))]).reshape(
      NS, NBLK64, BLK)

  zeros = jnp.zeros((N_PAD, 64), jnp.float32)
  ones_blk = jnp.ones((BLK, 16), jnp.float32)

  dp = _deg16(ones_blk, dst16, zeros)
  dinv, y1 = _tc_first(dp, x, W1)

  q = _agg64(y1.reshape(2 * N, 64), src64, dst64, zeros)
  y2 = _tc_mid(y1, q, dinv, b1.reshape(1, 128), W2)

  q = _agg64(y2.reshape(2 * N, 64), src64, dst64, zeros)
  w3p = jnp.pad(W3, ((0, 0), (0, 128 - W3.shape[1])))
  y3 = _tc_mid(y2, q, dinv, b2.reshape(1, 128), w3p)

  r = _agg16(y3.reshape(8 * N, 16), src16, dst16, zeros)
  return _tc_last(y3, r, dinv, b3.reshape(1, 2))
